# baseline (speedup 1.0000x reference)
import functools

import jax
import jax.numpy as jnp
from jax import lax
from jax.experimental import pallas as pl
from jax.experimental.pallas import tpu as pltpu
from jax.experimental.pallas import tpu_sc as plsc

D = 1024
L = 16
VPR = D // L

_info = plsc.get_sparse_core_info()
NC, NS = _info.num_cores, _info.num_subcores
NW = NC * NS


def kernel(x, encoding):
    B, S, d = x.shape
    assert d == D
    R = B * S
    rows_per_w = R // NW
    C = 32
    n_chunks = rows_per_w // C

    x2 = x.reshape(R, D)

    mesh = plsc.VectorSubcoreMesh(core_axis_name="c", subcore_axis_name="s")

    @functools.partial(
        pl.kernel,
        out_type=jax.ShapeDtypeStruct((R, D), jnp.float32),
        mesh=mesh,
        scratch_types=[
            pltpu.VMEM((2, D), jnp.float32),
            pltpu.VMEM((C, D), jnp.float32),
        ],
    )
    def k(x_hbm, enc_hbm, out_hbm, enc_v, buf):
        wid = lax.axis_index("s") * NC + lax.axis_index("c")
        wstart = wid * rows_per_w
        pltpu.sync_copy(enc_hbm.at[pl.ds(0, 2)], enc_v)

        def chunk_body(i, carry):
            base = wstart + i * C
            pltpu.sync_copy(x_hbm.at[pl.ds(base, C)], buf)

            def row_body(r, c2):
                for j in range(VPR):
                    sl = pl.ds(j * L, L)
                    buf[r, sl] += enc_v[0, sl]
                return c2

            lax.fori_loop(0, C, row_body, 0)
            pltpu.sync_copy(buf, out_hbm.at[pl.ds(base, C)])
            return carry

        lax.fori_loop(0, n_chunks, chunk_body, 0)

        last = wstart + rows_per_w - 1

        @pl.when((last + 1) % S == 0)
        def _fix():
            pltpu.sync_copy(x_hbm.at[pl.ds(last - 1, 1)], buf.at[pl.ds(0, 1)])
            for j in range(VPR):
                sl = pl.ds(j * L, L)
                buf[0, sl] += enc_v[1, sl]
            pltpu.sync_copy(buf.at[pl.ds(0, 1)], out_hbm.at[pl.ds(last, 1)])

    out = k(x2, encoding)
    return out.reshape(B, S, D)


# speedup vs baseline: 1.1658x; 1.1658x over previous
import functools

import jax
import jax.numpy as jnp
from jax import lax
from jax.experimental import pallas as pl
from jax.experimental.pallas import tpu as pltpu
from jax.experimental.pallas import tpu_sc as plsc

D = 1024
L = 16
VPR = D // L

_info = plsc.get_sparse_core_info()
NC, NS = _info.num_cores, _info.num_subcores
NW = NC * NS


def kernel(x, encoding):
    B, S, d = x.shape
    assert d == D
    R = B * S
    rows_per_w = R // NW
    C = 32
    n_chunks = rows_per_w // C

    x2 = x.reshape(R, D)

    mesh = plsc.VectorSubcoreMesh(core_axis_name="c", subcore_axis_name="s")

    @functools.partial(
        pl.kernel,
        out_type=jax.ShapeDtypeStruct((R, D), jnp.float32),
        mesh=mesh,
        scratch_types=[
            pltpu.VMEM((2, D), jnp.float32),
            pltpu.VMEM((C, D), jnp.float32),
            pltpu.VMEM((C, D), jnp.float32),
            pltpu.SemaphoreType.DMA,
            pltpu.SemaphoreType.DMA,
            pltpu.SemaphoreType.DMA,
            pltpu.SemaphoreType.DMA,
        ],
    )
    def k(x_hbm, enc_hbm, out_hbm, enc_v, buf0, buf1, is0, is1, os0, os1):
        bufs = (buf0, buf1)
        isems = (is0, is1)
        osems = (os0, os1)

        wid = lax.axis_index("s") * NC + lax.axis_index("c")
        wstart = wid * rows_per_w
        pltpu.sync_copy(enc_hbm.at[pl.ds(0, 2)], enc_v)

        def start_in(i, b):
            pltpu.async_copy(x_hbm.at[pl.ds(wstart + i * C, C)], bufs[b],
                             isems[b])

        def wait_in(b):
            pltpu.make_async_copy(x_hbm.at[pl.ds(0, C)], bufs[b],
                                  isems[b]).wait()

        def start_out(i, b):
            pltpu.async_copy(bufs[b], out_hbm.at[pl.ds(wstart + i * C, C)],
                             osems[b])

        def wait_out(b):
            pltpu.make_async_copy(bufs[b], out_hbm.at[pl.ds(0, C)],
                                  osems[b]).wait()

        def add_rows(buf):
            def row_body(r, c2):
                for j in range(VPR):
                    sl = pl.ds(j * L, L)
                    buf[r, sl] += enc_v[0, sl]
                return c2
            lax.fori_loop(0, C, row_body, 0)

        start_in(0, 0)
        start_in(1, 1)

        def outer(h, carry):
            for b in range(2):
                i = 2 * h + b
                wait_in(b)
                add_rows(bufs[b])
                start_out(i, b)

                @pl.when(i + 2 < n_chunks)
                def _prefetch():
                    wait_out(b)
                    start_in(i + 2, b)
            return carry

        lax.fori_loop(0, n_chunks // 2, outer, 0)
        wait_out(0)
        wait_out(1)

        last = wstart + rows_per_w - 1

        @pl.when((last + 1) % S == 0)
        def _fix():
            pltpu.sync_copy(x_hbm.at[pl.ds(last - 1, 1)], buf0.at[pl.ds(0, 1)])
            for j in range(VPR):
                sl = pl.ds(j * L, L)
                buf0[0, sl] += enc_v[1, sl]
            pltpu.sync_copy(buf0.at[pl.ds(0, 1)], out_hbm.at[pl.ds(last, 1)])

    out = k(x2, encoding)
    return out.reshape(B, S, D)


# speedup vs baseline: 3.0620x; 2.6266x over previous
import functools

import jax
import jax.numpy as jnp
from jax import lax
from jax.experimental import pallas as pl
from jax.experimental.pallas import tpu as pltpu
from jax.experimental.pallas import tpu_sc as plsc

D = 1024
L = 16
VPR = D // L

_info = plsc.get_sparse_core_info()
NC, NS = _info.num_cores, _info.num_subcores
NW = NC * NS


def kernel(x, encoding):
    B, S, d = x.shape
    assert d == D
    R = B * S
    rows_per_w = R // NW
    C = 32
    n_chunks = rows_per_w // C

    x2 = x.reshape(R, D)

    mesh = plsc.VectorSubcoreMesh(core_axis_name="c", subcore_axis_name="s")

    @functools.partial(
        pl.kernel,
        out_type=jax.ShapeDtypeStruct((R, D), jnp.float32),
        mesh=mesh,
        scratch_types=[
            pltpu.VMEM((2, D), jnp.float32),
            pltpu.VMEM((C, D), jnp.float32),
            pltpu.VMEM((C, D), jnp.float32),
            pltpu.SemaphoreType.DMA,
            pltpu.SemaphoreType.DMA,
            pltpu.SemaphoreType.DMA,
            pltpu.SemaphoreType.DMA,
        ],
    )
    def k(x_hbm, enc_hbm, out_hbm, enc_v, buf0, buf1, is0, is1, os0, os1):
        bufs = (buf0, buf1)
        isems = (is0, is1)
        osems = (os0, os1)

        wid = lax.axis_index("s") * NC + lax.axis_index("c")
        wstart = wid * rows_per_w
        pltpu.sync_copy(enc_hbm.at[pl.ds(0, 2)], enc_v)

        def start_in(i, b):
            pltpu.async_copy(x_hbm.at[pl.ds(wstart + i * C, C)], bufs[b],
                             isems[b])

        def wait_in(b):
            pltpu.make_async_copy(x_hbm.at[pl.ds(0, C)], bufs[b],
                                  isems[b]).wait()

        def start_out(i, b):
            pltpu.async_copy(bufs[b], out_hbm.at[pl.ds(wstart + i * C, C)],
                             osems[b])

        def wait_out(b):
            pltpu.make_async_copy(bufs[b], out_hbm.at[pl.ds(0, C)],
                                  osems[b]).wait()

        def add_rows(buf):
            H = VPR // 2
            for half in range(2):
                evs = [enc_v[0, pl.ds((half * H + j) * L, L)] for j in range(H)]

                def row_body(r):
                    for j in range(H):
                        buf[r, pl.ds((half * H + j) * L, L)] += evs[j]

                plsc.parallel_loop(0, C, 1, unroll=2)(row_body)

        start_in(0, 0)
        start_in(1, 1)

        def outer(h, carry):
            for b in range(2):
                i = 2 * h + b
                wait_in(b)
                add_rows(bufs[b])
                start_out(i, b)

                @pl.when(i + 2 < n_chunks)
                def _prefetch():
                    wait_out(b)
                    start_in(i + 2, b)
            return carry

        lax.fori_loop(0, n_chunks // 2, outer, 0)
        wait_out(0)
        wait_out(1)

        last = wstart + rows_per_w - 1

        @pl.when((last + 1) % S == 0)
        def _fix():
            pltpu.sync_copy(x_hbm.at[pl.ds(last - 1, 1)], buf0.at[pl.ds(0, 1)])
            for j in range(VPR):
                sl = pl.ds(j * L, L)
                buf0[0, sl] += enc_v[1, sl]
            pltpu.sync_copy(buf0.at[pl.ds(0, 1)], out_hbm.at[pl.ds(last, 1)])

    out = k(x2, encoding)
    return out.reshape(B, S, D)
